# baseline (speedup 1.0000x reference)
import functools

import numpy as np

import jax
import jax.numpy as jnp
from jax import lax
from jax.experimental import pallas as pl
from jax.experimental.pallas import tpu as pltpu
from jax.experimental.pallas import tpu_sc as plsc

_ATOMS = 51
_V_MAX = 1.0
_V_MIN = -1.0
_BATCH = 16384
_TC_ROWS = 8192
_SC_WORKERS = 32
_SC_ROWS = _BATCH - _TC_ROWS
_SC_ROWS_PW = _SC_ROWS // _SC_WORKERS

_LN2 = 0.6931471805599453
_SQRT2 = 1.4142135623730951


def _projection_matrix(batch_size: int) -> np.ndarray:
    atoms = _ATOMS
    delta = np.float32((_V_MAX - _V_MIN) / (atoms - 1))
    supports = np.linspace(_V_MIN, _V_MAX, atoms).astype(np.float32)
    tz = np.clip(supports, np.float32(_V_MIN), np.float32(_V_MAX))
    b = (tz - np.float32(_V_MIN)) / delta
    l = np.floor(b).astype(np.int32)
    u = np.ceil(b).astype(np.int32)
    l2 = np.where((u > 0) & (l == u), l - 1, l)
    u2 = np.where((l2 < atoms - 1) & (l2 == u), u + 1, u)
    wl = (u2.astype(np.float32) - b).astype(np.float32)
    wu = (b - l2.astype(np.float32)).astype(np.float32)
    p = np.zeros((atoms, atoms), dtype=np.float64)
    for j in range(atoms):
        p[j, l2[j]] += wl[j]
        p[j, u2[j]] += wu[j]
    return (p.astype(np.float32) * (-1.0 / batch_size)).astype(np.float32)


def _tc_body(p_ref, a_ref, f_ref, out_ref):
    logf = jnp.log(f_ref[...] + 1e-16)
    skewed = jax.lax.dot_general(
        a_ref[...], p_ref[...],
        dimension_numbers=(((1,), (0,)), ((), ())),
        preferred_element_type=jnp.float32,
    )
    part = jnp.sum(skewed * logf, axis=(0, 1), keepdims=True)

    @pl.when(pl.program_id(0) == 0)
    def _init():
        out_ref[...] = part

    @pl.when(pl.program_id(0) != 0)
    def _acc():
        out_ref[...] += part


def _ln(y):
    i = lax.bitcast_convert_type(y, jnp.int32)
    e = lax.shift_right_arithmetic(i, 23) - 127
    m = lax.bitcast_convert_type(
        (i & jnp.int32(0x007FFFFF)) | jnp.int32(0x3F800000), jnp.float32)
    big = m > jnp.float32(_SQRT2)
    m2 = jnp.where(big, m * jnp.float32(0.5), m)
    e2 = jnp.where(big, e + 1, e).astype(jnp.float32)
    z = m2 - jnp.float32(1.0)
    t = z / (m2 + jnp.float32(1.0))
    t2 = t * t
    p = t * (jnp.float32(2.0) + t2 * (jnp.float32(2.0 / 3.0) + t2 * (
        jnp.float32(2.0 / 5.0) + t2 * jnp.float32(2.0 / 7.0))))
    return e2 * jnp.float32(_LN2) + p


def _sc_body(a_hbm, f_hbm, out_hbm, a_v, f_v, acc_v):
    wid = lax.axis_index("s") * 2 + lax.axis_index("c")
    base = _TC_ROWS + wid * _SC_ROWS_PW
    pltpu.sync_copy(a_hbm.at[pl.ds(base, _SC_ROWS_PW)], a_v)
    pltpu.sync_copy(f_hbm.at[pl.ds(base, _SC_ROWS_PW)], f_v)

    lanes = lax.iota(jnp.int32, 16)
    tail_keep = lanes >= 13

    def row_step(r, acc):
        for c0 in (0, 16, 32, 35):
            a = a_v[r, pl.ds(c0, 16)]
            f = f_v[r, pl.ds(c0, 16)]
            term = a * _ln(f + jnp.float32(1e-16))
            if c0 == 35:
                term = jnp.where(tail_keep, term, jnp.float32(0.0))
            acc = acc + term
        return acc

    acc = lax.fori_loop(0, _SC_ROWS_PW, row_step,
                        jnp.zeros((16,), jnp.float32))
    acc_v[...] = acc * jnp.float32(-1.0 / _BATCH)
    pltpu.sync_copy(acc_v, out_hbm.at[wid])


def _sc_call():
    return functools.partial(
        pl.kernel,
        out_type=jax.ShapeDtypeStruct((_SC_WORKERS, 16), jnp.float32),
        mesh=plsc.VectorSubcoreMesh(core_axis_name="c", subcore_axis_name="s"),
        scratch_types=[
            pltpu.VMEM((_SC_ROWS_PW, _ATOMS), jnp.float32),
            pltpu.VMEM((_SC_ROWS_PW, _ATOMS), jnp.float32),
            pltpu.VMEM((16,), jnp.float32),
        ],
        compiler_params=pltpu.CompilerParams(use_tc_tiling_on_sc=True),
    )


def kernel(anchor, feature):
    batch, atoms = anchor.shape
    proj = jnp.asarray(_projection_matrix(batch))
    block_b = 4096
    tc_part = pl.pallas_call(
        _tc_body,
        grid=(_TC_ROWS // block_b,),
        in_specs=[
            pl.BlockSpec((atoms, atoms), lambda i: (0, 0)),
            pl.BlockSpec((block_b, atoms), lambda i: (i, 0)),
            pl.BlockSpec((block_b, atoms), lambda i: (i, 0)),
        ],
        out_specs=pl.BlockSpec((1, 1), lambda i: (0, 0)),
        out_shape=jax.ShapeDtypeStruct((1, 1), jnp.float32),
    )(proj, anchor, feature)
    sc_part = _sc_call()(_sc_body)(anchor, feature)
    return tc_part[0, 0] + jnp.sum(sc_part)


# speedup vs baseline: 2.0315x; 2.0315x over previous
import numpy as np

import jax
import jax.numpy as jnp
from jax.experimental import pallas as pl

_ATOMS = 51
_V_MAX = 1.0
_V_MIN = -1.0


def _projection_matrix(batch_size: int) -> np.ndarray:
    atoms = _ATOMS
    delta = np.float32((_V_MAX - _V_MIN) / (atoms - 1))
    supports = np.linspace(_V_MIN, _V_MAX, atoms).astype(np.float32)
    tz = np.clip(supports, np.float32(_V_MIN), np.float32(_V_MAX))
    b = (tz - np.float32(_V_MIN)) / delta
    l = np.floor(b).astype(np.int32)
    u = np.ceil(b).astype(np.int32)
    l2 = np.where((u > 0) & (l == u), l - 1, l)
    u2 = np.where((l2 < atoms - 1) & (l2 == u), u + 1, u)
    wl = (u2.astype(np.float32) - b).astype(np.float32)
    wu = (b - l2.astype(np.float32)).astype(np.float32)
    p = np.zeros((atoms, atoms), dtype=np.float64)
    for j in range(atoms):
        p[j, l2[j]] += wl[j]
        p[j, u2[j]] += wu[j]
    return (p.astype(np.float32) * (-1.0 / batch_size)).astype(np.float32)


def _loss_body(p_ref, a_ref, f_ref, out_ref):
    logf = jnp.log(f_ref[...] + 1e-16)
    skewed = jax.lax.dot_general(
        a_ref[...], p_ref[...],
        dimension_numbers=(((1,), (0,)), ((), ())),
        preferred_element_type=jnp.float32,
    )
    part = jnp.sum(skewed * logf, axis=(0, 1), keepdims=True)

    @pl.when(pl.program_id(0) == 0)
    def _init():
        out_ref[...] = part

    @pl.when(pl.program_id(0) != 0)
    def _acc():
        out_ref[...] += part


def kernel(anchor, feature):
    batch, atoms = anchor.shape
    proj = jnp.asarray(_projection_matrix(batch))
    block_b = 8192
    grid = (batch // block_b,)
    out = pl.pallas_call(
        _loss_body,
        grid=grid,
        in_specs=[
            pl.BlockSpec((atoms, atoms), lambda i: (0, 0)),
            pl.BlockSpec((block_b, atoms), lambda i: (i, 0)),
            pl.BlockSpec((block_b, atoms), lambda i: (i, 0)),
        ],
        out_specs=pl.BlockSpec((1, 1), lambda i: (0, 0)),
        out_shape=jax.ShapeDtypeStruct((1, 1), jnp.float32),
    )(proj, anchor, feature)
    return out[0, 0]


# speedup vs baseline: 2.0520x; 1.0101x over previous
import numpy as np

import jax
import jax.numpy as jnp
from jax.experimental import pallas as pl

_ATOMS = 51
_V_MAX = 1.0
_V_MIN = -1.0


def _projection_matrix(batch_size: int) -> np.ndarray:
    atoms = _ATOMS
    delta = np.float32((_V_MAX - _V_MIN) / (atoms - 1))
    supports = np.linspace(_V_MIN, _V_MAX, atoms).astype(np.float32)
    tz = np.clip(supports, np.float32(_V_MIN), np.float32(_V_MAX))
    b = (tz - np.float32(_V_MIN)) / delta
    l = np.floor(b).astype(np.int32)
    u = np.ceil(b).astype(np.int32)
    l2 = np.where((u > 0) & (l == u), l - 1, l)
    u2 = np.where((l2 < atoms - 1) & (l2 == u), u + 1, u)
    wl = (u2.astype(np.float32) - b).astype(np.float32)
    wu = (b - l2.astype(np.float32)).astype(np.float32)
    p = np.zeros((atoms, atoms), dtype=np.float64)
    for j in range(atoms):
        p[j, l2[j]] += wl[j]
        p[j, u2[j]] += wu[j]
    return (p.astype(np.float32) * (-1.0 / batch_size)).astype(np.float32)


def _loss_body(p_ref, a_ref, f_ref, out_ref):
    logf = jnp.log(f_ref[...] + 1e-16)
    skewed = jax.lax.dot_general(
        a_ref[...], p_ref[...],
        dimension_numbers=(((1,), (0,)), ((), ())),
        preferred_element_type=jnp.float32,
    )
    part = jnp.sum(skewed * logf, axis=(0, 1), keepdims=True)

    @pl.when(pl.program_id(0) == 0)
    def _init():
        out_ref[...] = part

    @pl.when(pl.program_id(0) != 0)
    def _acc():
        out_ref[...] += part


def kernel(anchor, feature):
    batch, atoms = anchor.shape
    proj = jnp.asarray(_projection_matrix(batch))
    block_b = 16384
    grid = (batch // block_b,)
    out = pl.pallas_call(
        _loss_body,
        grid=grid,
        in_specs=[
            pl.BlockSpec((atoms, atoms), lambda i: (0, 0)),
            pl.BlockSpec((block_b, atoms), lambda i: (i, 0)),
            pl.BlockSpec((block_b, atoms), lambda i: (i, 0)),
        ],
        out_specs=pl.BlockSpec((1, 1), lambda i: (0, 0)),
        out_shape=jax.ShapeDtypeStruct((1, 1), jnp.float32),
    )(proj, anchor, feature)
    return out[0, 0]


# speedup vs baseline: 2.0791x; 1.0132x over previous
import numpy as np

import jax
import jax.numpy as jnp
from jax.experimental import pallas as pl

_ATOMS = 51
_V_MAX = 1.0
_V_MIN = -1.0


def _projection_matrix(batch_size: int) -> np.ndarray:
    atoms = _ATOMS
    delta = np.float32((_V_MAX - _V_MIN) / (atoms - 1))
    supports = np.linspace(_V_MIN, _V_MAX, atoms).astype(np.float32)
    tz = np.clip(supports, np.float32(_V_MIN), np.float32(_V_MAX))
    b = (tz - np.float32(_V_MIN)) / delta
    l = np.floor(b).astype(np.int32)
    u = np.ceil(b).astype(np.int32)
    l2 = np.where((u > 0) & (l == u), l - 1, l)
    u2 = np.where((l2 < atoms - 1) & (l2 == u), u + 1, u)
    wl = (u2.astype(np.float32) - b).astype(np.float32)
    wu = (b - l2.astype(np.float32)).astype(np.float32)
    p = np.zeros((atoms, atoms), dtype=np.float64)
    for j in range(atoms):
        p[j, l2[j]] += wl[j]
        p[j, u2[j]] += wu[j]
    return (p.astype(np.float32) * (-1.0 / batch_size)).astype(np.float32)


def _loss_body(p_ref, a1_ref, a2_ref, f1_ref, f2_ref, out_ref):
    p = p_ref[...]

    def half(a_ref, f_ref):
        logf = jnp.log(f_ref[...] + 1e-16)
        skewed = jax.lax.dot_general(
            a_ref[...], p,
            dimension_numbers=(((1,), (0,)), ((), ())),
            preferred_element_type=jnp.float32,
        )
        return jnp.sum(skewed * logf, axis=(0, 1), keepdims=True)

    out_ref[...] = half(a1_ref, f1_ref) + half(a2_ref, f2_ref)


def kernel(anchor, feature):
    batch, atoms = anchor.shape
    proj = jnp.asarray(_projection_matrix(batch))
    half_b = batch // 2
    out = pl.pallas_call(
        _loss_body,
        grid=(1,),
        in_specs=[
            pl.BlockSpec((atoms, atoms), lambda i: (0, 0)),
            pl.BlockSpec((half_b, atoms), lambda i: (0, 0)),
            pl.BlockSpec((half_b, atoms), lambda i: (1, 0)),
            pl.BlockSpec((half_b, atoms), lambda i: (0, 0)),
            pl.BlockSpec((half_b, atoms), lambda i: (1, 0)),
        ],
        out_specs=pl.BlockSpec((1, 1), lambda i: (0, 0)),
        out_shape=jax.ShapeDtypeStruct((1, 1), jnp.float32),
    )(proj, anchor, anchor, feature, feature)
    return out[0, 0]


# speedup vs baseline: 2.0874x; 1.0040x over previous
import numpy as np

import jax
import jax.numpy as jnp
from jax.experimental import pallas as pl

_ATOMS = 51
_V_MAX = 1.0
_V_MIN = -1.0


def _projection_matrix(batch_size: int) -> np.ndarray:
    atoms = _ATOMS
    delta = np.float32((_V_MAX - _V_MIN) / (atoms - 1))
    supports = np.linspace(_V_MIN, _V_MAX, atoms).astype(np.float32)
    tz = np.clip(supports, np.float32(_V_MIN), np.float32(_V_MAX))
    b = (tz - np.float32(_V_MIN)) / delta
    l = np.floor(b).astype(np.int32)
    u = np.ceil(b).astype(np.int32)
    l2 = np.where((u > 0) & (l == u), l - 1, l)
    u2 = np.where((l2 < atoms - 1) & (l2 == u), u + 1, u)
    wl = (u2.astype(np.float32) - b).astype(np.float32)
    wu = (b - l2.astype(np.float32)).astype(np.float32)
    p = np.zeros((atoms, atoms), dtype=np.float64)
    for j in range(atoms):
        p[j, l2[j]] += wl[j]
        p[j, u2[j]] += wu[j]
    return (p.astype(np.float32) * (-1.0 / batch_size)).astype(np.float32)


_N_STREAMS = 4


def _loss_body(p_ref, *refs):
    a_refs = refs[:_N_STREAMS]
    f_refs = refs[_N_STREAMS:2 * _N_STREAMS]
    out_ref = refs[2 * _N_STREAMS]
    p = p_ref[...]

    def piece(a_ref, f_ref):
        logf = jnp.log(f_ref[...] + 1e-16)
        skewed = jax.lax.dot_general(
            a_ref[...], p,
            dimension_numbers=(((1,), (0,)), ((), ())),
            preferred_element_type=jnp.float32,
        )
        return jnp.sum(skewed * logf, axis=(0, 1), keepdims=True)

    out_ref[...] = sum(piece(a, f) for a, f in zip(a_refs, f_refs))


def kernel(anchor, feature):
    batch, atoms = anchor.shape
    proj = jnp.asarray(_projection_matrix(batch))
    part_b = batch // _N_STREAMS

    def spec(k):
        return pl.BlockSpec((part_b, atoms), lambda i, _k=k: (_k, 0))

    out = pl.pallas_call(
        _loss_body,
        grid=(1,),
        in_specs=[pl.BlockSpec((atoms, atoms), lambda i: (0, 0))]
        + [spec(k) for k in range(_N_STREAMS)] * 2,
        out_specs=pl.BlockSpec((1, 1), lambda i: (0, 0)),
        out_shape=jax.ShapeDtypeStruct((1, 1), jnp.float32),
    )(proj, *([anchor] * _N_STREAMS), *([feature] * _N_STREAMS))
    return out[0, 0]


# speedup vs baseline: 2.0877x; 1.0002x over previous
import numpy as np

import jax
import jax.numpy as jnp
from jax.experimental import pallas as pl

_ATOMS = 51
_V_MAX = 1.0
_V_MIN = -1.0


def _projection_matrix(batch_size: int) -> np.ndarray:
    atoms = _ATOMS
    delta = np.float32((_V_MAX - _V_MIN) / (atoms - 1))
    supports = np.linspace(_V_MIN, _V_MAX, atoms).astype(np.float32)
    tz = np.clip(supports, np.float32(_V_MIN), np.float32(_V_MAX))
    b = (tz - np.float32(_V_MIN)) / delta
    l = np.floor(b).astype(np.int32)
    u = np.ceil(b).astype(np.int32)
    l2 = np.where((u > 0) & (l == u), l - 1, l)
    u2 = np.where((l2 < atoms - 1) & (l2 == u), u + 1, u)
    wl = (u2.astype(np.float32) - b).astype(np.float32)
    wu = (b - l2.astype(np.float32)).astype(np.float32)
    p = np.zeros((atoms, atoms), dtype=np.float64)
    for j in range(atoms):
        p[j, l2[j]] += wl[j]
        p[j, u2[j]] += wu[j]
    return (p.astype(np.float32) * (-1.0 / batch_size)).astype(np.float32)


_N_STREAMS = 8


def _loss_body(p_ref, *refs):
    a_refs = refs[:_N_STREAMS]
    f_refs = refs[_N_STREAMS:2 * _N_STREAMS]
    out_ref = refs[2 * _N_STREAMS]
    p = p_ref[...]

    def piece(a_ref, f_ref):
        logf = jnp.log(f_ref[...] + 1e-16)
        skewed = jax.lax.dot_general(
            a_ref[...], p,
            dimension_numbers=(((1,), (0,)), ((), ())),
            preferred_element_type=jnp.float32,
        )
        return jnp.sum(skewed * logf, axis=(0, 1), keepdims=True)

    out_ref[...] = sum(piece(a, f) for a, f in zip(a_refs, f_refs))


def kernel(anchor, feature):
    batch, atoms = anchor.shape
    proj = jnp.asarray(_projection_matrix(batch))
    part_b = batch // _N_STREAMS

    def spec(k):
        return pl.BlockSpec((part_b, atoms), lambda i, _k=k: (_k, 0))

    out = pl.pallas_call(
        _loss_body,
        grid=(1,),
        in_specs=[pl.BlockSpec((atoms, atoms), lambda i: (0, 0))]
        + [spec(k) for k in range(_N_STREAMS)] * 2,
        out_specs=pl.BlockSpec((1, 1), lambda i: (0, 0)),
        out_shape=jax.ShapeDtypeStruct((1, 1), jnp.float32),
    )(proj, *([anchor] * _N_STREAMS), *([feature] * _N_STREAMS))
    return out[0, 0]


# speedup vs baseline: 2.0943x; 1.0032x over previous
import numpy as np

import jax
import jax.numpy as jnp
from jax.experimental import pallas as pl

_ATOMS = 51
_V_MAX = 1.0
_V_MIN = -1.0


def _projection_matrix(batch_size: int) -> np.ndarray:
    atoms = _ATOMS
    delta = np.float32((_V_MAX - _V_MIN) / (atoms - 1))
    supports = np.linspace(_V_MIN, _V_MAX, atoms).astype(np.float32)
    tz = np.clip(supports, np.float32(_V_MIN), np.float32(_V_MAX))
    b = (tz - np.float32(_V_MIN)) / delta
    l = np.floor(b).astype(np.int32)
    u = np.ceil(b).astype(np.int32)
    l2 = np.where((u > 0) & (l == u), l - 1, l)
    u2 = np.where((l2 < atoms - 1) & (l2 == u), u + 1, u)
    wl = (u2.astype(np.float32) - b).astype(np.float32)
    wu = (b - l2.astype(np.float32)).astype(np.float32)
    p = np.zeros((atoms, atoms), dtype=np.float64)
    for j in range(atoms):
        p[j, l2[j]] += wl[j]
        p[j, u2[j]] += wu[j]
    return (p.astype(np.float32) * (-1.0 / batch_size)).astype(np.float32)


_N_STREAMS = 4


def _loss_body(p_ref, *refs):
    a_refs = refs[:_N_STREAMS]
    f_refs = refs[_N_STREAMS:2 * _N_STREAMS]
    out_ref = refs[2 * _N_STREAMS]
    p = p_ref[...]

    def piece(a_ref, f_ref):
        logf = jnp.log(f_ref[...] + 1e-16)
        skewed = jax.lax.dot_general(
            a_ref[...], p,
            dimension_numbers=(((1,), (0,)), ((), ())),
            preferred_element_type=jnp.float32,
        )
        return jnp.sum(skewed * logf, axis=(0, 1), keepdims=True)

    out_ref[...] = sum(piece(a, f) for a, f in zip(a_refs, f_refs))


def kernel(anchor, feature):
    batch, atoms = anchor.shape
    proj = jnp.asarray(_projection_matrix(batch))
    part_b = batch // _N_STREAMS

    def spec(k):
        return pl.BlockSpec((part_b, atoms), lambda i, _k=k: (_k, 0))

    out = pl.pallas_call(
        _loss_body,
        grid=(1,),
        in_specs=[pl.BlockSpec((atoms, atoms), lambda i: (0, 0))]
        + [spec(k) for k in range(_N_STREAMS)] * 2,
        out_specs=pl.BlockSpec((1, 1), lambda i: (0, 0)),
        out_shape=jax.ShapeDtypeStruct((1, 1), jnp.float32),
    )(proj, *([anchor] * _N_STREAMS), *([feature] * _N_STREAMS))
    return out[0, 0]
